# baseline (speedup 1.0000x reference)
import functools

import jax
import jax.numpy as jnp
from jax import lax
from jax.experimental import pallas as pl
from jax.experimental.pallas import tpu as pltpu
from jax.experimental.pallas import tpu_sc as plsc

VOCAB_SIZE = 1_000_000
EMBED_DIM = 64
BATCH = 4096
SEQ_LEN = 200
MAX_WAVELENGTH = 10000.0

NUM_CORES = 2
NUM_SUBCORES = 16
NW = NUM_CORES * NUM_SUBCORES
TPW = BATCH * SEQ_LEN // NW
CHUNK = 128
NCHUNK = TPW // CHUNK
PACKED = CHUNK // 2
NBUF = 4
LANES = 16


def _pos_encoding():
    position = jnp.arange(SEQ_LEN, dtype=jnp.float32)
    min_freq = 1.0 / MAX_WAVELENGTH
    timescales = jnp.power(
        min_freq,
        (2.0 * (jnp.arange(EMBED_DIM, dtype=jnp.float32) // 2)) / float(EMBED_DIM),
    )
    angles = position[:, None] * timescales[None, :]
    cos_mask = jnp.asarray(jnp.arange(EMBED_DIM) % 2, dtype=jnp.float32)
    sin_mask = 1.0 - cos_mask
    return jnp.sin(angles) * sin_mask + jnp.cos(angles) * cos_mask


_mesh = plsc.VectorSubcoreMesh(core_axis_name="c", subcore_axis_name="s")


@functools.partial(
    pl.kernel,
    out_type=jax.ShapeDtypeStruct((BATCH * SEQ_LEN // 2, 2 * EMBED_DIM), jnp.float32),
    mesh=_mesh,
    compiler_params=pltpu.CompilerParams(use_tc_tiling_on_sc=False),
    scratch_types=(
        [pltpu.VMEM((CHUNK, 2 * EMBED_DIM), jnp.float32) for _ in range(NBUF)]
        + [pltpu.VMEM((CHUNK,), jnp.int32) for _ in range(NBUF)]
        + [
            pltpu.VMEM((NCHUNK, CHUNK), jnp.int32),
            pltpu.VMEM((SEQ_LEN, EMBED_DIM), jnp.float32),
        ]
        + [pltpu.SemaphoreType.DMA] * NBUF
        + [pltpu.SemaphoreType.DMA] * NBUF
    ),
)
def _emb_kernel(x_hbm, table_hbm, pos_hbm, out_hbm, *scratch):
    rows = scratch[:NBUF]
    pidx = scratch[NBUF : 2 * NBUF]
    idx_v = scratch[2 * NBUF]
    pos_v = scratch[2 * NBUF + 1]
    sg = scratch[2 * NBUF + 2 : 2 * NBUF + 2 + NBUF]
    so = scratch[2 * NBUF + 2 + NBUF :]

    wid = lax.axis_index("s") * NUM_CORES + lax.axis_index("c")
    obase = wid * (TPW // 2)
    pltpu.sync_copy(x_hbm.at[wid], idx_v)
    pltpu.sync_copy(pos_hbm, pos_v)

    def fire_gather(t, a):
        for q in range(CHUNK // LANES):
            sl = pl.ds(q * LANES, LANES)
            pidx[a][sl] = lax.shift_right_logical(idx_v[t, sl], 1)
        pltpu.async_copy(table_hbm.at[pidx[a]], rows[a], sg[a])

    def wait_gather(a):
        pltpu.make_async_copy(table_hbm.at[pidx[a]], rows[a], sg[a]).wait()

    def wait_out(a):
        pltpu.make_async_copy(
            rows[a].at[pl.ds(0, PACKED)], out_hbm.at[pl.ds(obase, PACKED)], so[a]
        ).wait()

    fire_gather(0, 0)
    fire_gather(1, 1)

    def outer(tt, carry):
        for a in range(NBUF):
            t = tt * NBUF + a

            @pl.when(t >= NBUF)
            def _():
                wait_out(a)

            wait_gather(a)
            pbase = lax.rem(t * CHUNK, SEQ_LEN)

            def add_pack(g, c):
                vvec = idx_v[t, pl.ds(g * LANES, LANES)]
                for j in range(LANES):
                    k = g * LANES + j
                    kk = g * (LANES // 2) + j // 2
                    off = (vvec[j] & 1) * EMBED_DIM
                    l = pbase + k
                    l = lax.select(l >= SEQ_LEN, l - SEQ_LEN, l)
                    for q in range(EMBED_DIM // LANES):
                        rows[a][kk, pl.ds((j % 2) * EMBED_DIM + q * LANES, LANES)] = (
                            rows[a][k, pl.ds(off + q * LANES, LANES)]
                            + pos_v[l, pl.ds(q * LANES, LANES)]
                        )
                return c

            lax.fori_loop(0, CHUNK // LANES, add_pack, 0)
            pltpu.async_copy(
                rows[a].at[pl.ds(0, PACKED)],
                out_hbm.at[pl.ds(obase + t * PACKED, PACKED)],
                so[a],
            )

            tn = t + 2

            @pl.when(tn < NCHUNK)
            def _():
                fire_gather(tn, (a + 2) % NBUF)

        return carry

    lax.fori_loop(0, NCHUNK // NBUF, outer, 0)

    for a in range(NBUF):
        wait_out(a)


def kernel(x, token_emb_table):
    pos = _pos_encoding()
    x_r = x.astype(jnp.int32).reshape(NW, NCHUNK, CHUNK)
    table2 = token_emb_table.reshape(VOCAB_SIZE // 2, 2 * EMBED_DIM)
    out = _emb_kernel(x_r, table2, pos)
    return out.reshape(BATCH, SEQ_LEN, EMBED_DIM)
